# baseline (speedup 1.0000x reference)
import functools

import jax
import jax.numpy as jnp
from jax import lax
from jax.experimental import pallas as pl
from jax.experimental.pallas import tpu as pltpu
from jax.experimental.pallas import tpu_sc as plsc

N_STEPS = 20
B = 1024
V = 100000
NCORE = 2
NSUB = 16
NW = NCORE * NSUB
LANES = 16
SW = 3168
R = 24
NP = SW // R
NH = 2
NBG = B // LANES


def _body(probs_hbm, bits_hbm, step_hbm, out_hbm, *refs):
    step_v, sh_v, bits_v = refs[:3]
    bufs = refs[3:3 + 2 * NH]
    sems = refs[3 + 2 * NH:]

    cid = lax.axis_index("c")
    sid = lax.axis_index("s")
    wid = sid * NCORE + cid
    v0 = jnp.minimum(wid * SW, V - SW)

    pltpu.sync_copy(step_hbm, step_v)
    pltpu.sync_copy(bits_hbm.at[pl.ds(v0, SW)], bits_v)

    def sh_xform(g, _):
        sh_v[pl.ds(g * LANES, LANES)] = 32 - step_v[pl.ds(g * LANES, LANES)]
        return 0

    lax.fori_loop(0, NBG, sh_xform, 0)

    def in_copy(p, buf, sem):
        pltpu.async_copy(
            probs_hbm.at[pl.ds(v0 + p * R, R), pl.ds(0, B)], buf, sem)

    def out_copy(p, buf, sem):
        pltpu.async_copy(
            buf, out_hbm.at[pl.ds(v0 + p * R, R), pl.ds(0, B)], sem)

    halves = [(bufs[2 * h], bufs[2 * h + 1], sems[2 * h], sems[2 * h + 1])
              for h in range(NH)]

    for h, (ibuf, obuf, isem, osem) in enumerate(halves):
        in_copy(h, ibuf, isem)

    def round_body(pp, _):
        for h, (ibuf, obuf, isem, osem) in enumerate(halves):
            p = pp * NH + h

            def half(ibuf=ibuf, obuf=obuf, isem=isem, osem=osem, p=p):
                pltpu.make_async_copy(
                    probs_hbm.at[pl.ds(0, R), pl.ds(0, B)],
                    ibuf, isem).wait()

                @pl.when(pp > 0)
                def _():
                    pltpu.make_async_copy(
                        obuf, out_hbm.at[pl.ds(0, R), pl.ds(0, B)],
                        osem).wait()

                mbvec = bits_v[pl.ds(p * R, R)]
                words = [jnp.broadcast_to(mbvec[r], (LANES,))
                         for r in range(R)]

                @plsc.parallel_loop(0, NBG, 1)
                def _(bg):
                    boff = bg * LANES
                    sh = sh_v[pl.ds(boff, LANES)]
                    for r in range(R):
                        t = words[r] << sh
                        pv = ibuf[r, pl.ds(boff, LANES)]
                        obuf[r, pl.ds(boff, LANES)] = jnp.where(
                            t < 0, pv, 0.0)

                out_copy(p, obuf, osem)

                @pl.when(pp < NP // NH - 1)
                def _():
                    in_copy(p + NH, ibuf, isem)

            half()
        return 0

    lax.fori_loop(0, NP // NH, round_body, 0)

    for h, (ibuf, obuf, isem, osem) in enumerate(halves):
        pltpu.make_async_copy(
            obuf, out_hbm.at[pl.ds(0, R), pl.ds(0, B)], osem).wait()


def kernel(probabilites, mask, step):
    bits = jnp.sum(
        jnp.where(mask != 0, 1, 0).astype(jnp.int32)
        << jnp.arange(N_STEPS, dtype=jnp.int32)[:, None],
        axis=0)
    probs_t = probabilites.T
    mesh = plsc.VectorSubcoreMesh(core_axis_name="c", subcore_axis_name="s")
    scratch = [
        pltpu.VMEM((B,), jnp.int32),
        pltpu.VMEM((B,), jnp.int32),
        pltpu.VMEM((SW,), jnp.int32),
    ]
    scratch += [pltpu.VMEM((R, B), jnp.float32)] * (2 * NH)
    scratch += [pltpu.SemaphoreType.DMA] * (2 * NH)
    f = pl.kernel(
        _body,
        mesh=mesh,
        out_type=jax.ShapeDtypeStruct((V, B), jnp.float32),
        scratch_types=scratch,
    )
    return f(probs_t, bits, step).T


# speedup vs baseline: 1.0122x; 1.0122x over previous
import functools

import jax
import jax.numpy as jnp
from jax import lax
from jax.experimental import pallas as pl
from jax.experimental.pallas import tpu as pltpu
from jax.experimental.pallas import tpu_sc as plsc

N_STEPS = 20
B = 1024
V = 100000
NCORE = 2
NSUB = 16
NW = NCORE * NSUB
LANES = 16
SW = 3136
R = 16
NP = SW // R
NH = 2
NBG = B // LANES


def _body(probs_hbm, bits_hbm, step_hbm, out_hbm, *refs):
    step_v, sh_v, bits_v = refs[:3]
    bufs = refs[3:3 + 2 * NH]
    sems = refs[3 + 2 * NH:]

    cid = lax.axis_index("c")
    sid = lax.axis_index("s")
    wid = sid * NCORE + cid
    v0 = jnp.minimum(wid * SW, V - SW)

    pltpu.sync_copy(step_hbm, step_v)
    pltpu.sync_copy(bits_hbm.at[pl.ds(v0, SW)], bits_v)

    def sh_xform(g, _):
        sh_v[pl.ds(g * LANES, LANES)] = 32 - step_v[pl.ds(g * LANES, LANES)]
        return 0

    lax.fori_loop(0, NBG, sh_xform, 0)

    def in_copy(p, buf, sem):
        pltpu.async_copy(
            probs_hbm.at[pl.ds(v0 + p * R, R), pl.ds(0, B)], buf, sem)

    def out_copy(p, buf, sem):
        pltpu.async_copy(
            buf, out_hbm.at[pl.ds(v0 + p * R, R), pl.ds(0, B)], sem)

    halves = [(bufs[2 * h], bufs[2 * h + 1], sems[2 * h], sems[2 * h + 1])
              for h in range(NH)]

    for h, (ibuf, obuf, isem, osem) in enumerate(halves):
        in_copy(h, ibuf, isem)

    def round_body(pp, _):
        for h, (ibuf, obuf, isem, osem) in enumerate(halves):
            p = pp * NH + h

            def half(ibuf=ibuf, obuf=obuf, isem=isem, osem=osem, p=p):
                pltpu.make_async_copy(
                    probs_hbm.at[pl.ds(0, R), pl.ds(0, B)],
                    ibuf, isem).wait()

                @pl.when(pp > 0)
                def _():
                    pltpu.make_async_copy(
                        obuf, out_hbm.at[pl.ds(0, R), pl.ds(0, B)],
                        osem).wait()

                mbvec = bits_v[pl.ds(p * R, R)]
                words = [jnp.broadcast_to(mbvec[r], (LANES,))
                         for r in range(R)]

                @plsc.parallel_loop(0, NBG, 1)
                def _(bg):
                    boff = bg * LANES
                    sh = sh_v[pl.ds(boff, LANES)]
                    for r in range(R):
                        t = words[r] << sh
                        pv = ibuf[r, pl.ds(boff, LANES)]
                        obuf[r, pl.ds(boff, LANES)] = jnp.where(
                            t < 0, pv, 0.0)

                out_copy(p, obuf, osem)

                @pl.when(pp < NP // NH - 1)
                def _():
                    in_copy(p + NH, ibuf, isem)

            half()
        return 0

    lax.fori_loop(0, NP // NH, round_body, 0)

    for h, (ibuf, obuf, isem, osem) in enumerate(halves):
        pltpu.make_async_copy(
            obuf, out_hbm.at[pl.ds(0, R), pl.ds(0, B)], osem).wait()


def kernel(probabilites, mask, step):
    bits = jnp.sum(
        jnp.where(mask != 0, 1, 0).astype(jnp.int32)
        << jnp.arange(N_STEPS, dtype=jnp.int32)[:, None],
        axis=0)
    probs_t = probabilites.T
    mesh = plsc.VectorSubcoreMesh(core_axis_name="c", subcore_axis_name="s")
    scratch = [
        pltpu.VMEM((B,), jnp.int32),
        pltpu.VMEM((B,), jnp.int32),
        pltpu.VMEM((SW,), jnp.int32),
    ]
    scratch += [pltpu.VMEM((R, B), jnp.float32)] * (2 * NH)
    scratch += [pltpu.SemaphoreType.DMA] * (2 * NH)
    f = pl.kernel(
        _body,
        mesh=mesh,
        out_type=jax.ShapeDtypeStruct((V, B), jnp.float32),
        scratch_types=scratch,
    )
    return f(probs_t, bits, step).T
